# baseline (speedup 1.0000x reference)
import functools

import jax
import jax.numpy as jnp
from jax import lax
from jax.experimental import pallas as pl
from jax.experimental.pallas import tpu as pltpu
from jax.experimental.pallas import tpu_sc as plsc

P0, P1, P2 = 73856093, 19349663, 83492791
TABLE = 1 << 20
N = 500000
D = 16

NW = 32
BPW = TABLE // NW
QB = 2048
VPB = QB // 16
CPT = 15632
HMASK = TABLE - 1

_mesh = plsc.VectorSubcoreMesh(core_axis_name="c", subcore_axis_name="s")
_params = pltpu.CompilerParams(
    needs_layout_passes=False, use_tc_tiling_on_sc=False
)


_I = jnp.int32


def _wid():
    return lax.axis_index("s") * _I(2) + lax.axis_index("c")


def _tile_range(wid):
    i0 = wid * _I(CPT)
    cnt = jnp.minimum(_I(CPT), _I(N) - i0)
    return i0, cnt


SEG_CAP = 1024
SEGW = NW * SEG_CAP
SLACK = 2048
SENT = 0x7FFFFFFF


@functools.partial(
    pl.kernel,
    out_type=[
        jax.ShapeDtypeStruct((N,), jnp.int32),
        jax.ShapeDtypeStruct((NW * SEGW,), jnp.int32),
        jax.ShapeDtypeStruct((NW * SEGW,), jnp.int32),
    ],
    mesh=_mesh,
    compiler_params=_params,
    scratch_types=[
        pltpu.VMEM((QB,), jnp.int32),
        pltpu.VMEM((QB,), jnp.int32),
        pltpu.VMEM((QB,), jnp.int32),
        pltpu.VMEM((QB,), jnp.int32),
        pltpu.VMEM((SEGW + SLACK,), jnp.int32),
        pltpu.VMEM((SEGW + SLACK,), jnp.int32),
        pltpu.VMEM((32,), jnp.int32),
        pltpu.SemaphoreType.DMA,
    ],
)
def _route_k(xyz_hbm, h_hbm, segh_hbm, segj_hbm,
             xv, yv, zv, hv, segh_v, segj_v, cnt_v, sem):
    wid = _wid()
    i0, cnt = _tile_range(wid)
    trips = (cnt + _I(QB - 1)) // _I(QB)
    lanes = lax.iota(jnp.int32, 16)

    sent = jnp.full((16,), SENT, jnp.int32)

    def sinit(k, carry):
        segh_v[pl.ds(k * _I(16), 16)] = sent
        return carry

    lax.fori_loop(_I(0), _I((SEGW + SLACK) // 16), sinit, _I(0))
    cnt_v[pl.ds(_I(0), 16)] = jnp.zeros((16,), jnp.int32)
    cnt_v[pl.ds(_I(16), 16)] = jnp.zeros((16,), jnp.int32)

    p0 = jnp.full((16,), P0, jnp.int32)
    p1 = jnp.full((16,), P1, jnp.int32)
    p2 = jnp.full((16,), P2, jnp.int32)
    hm = jnp.full((16,), HMASK, jnp.int32)
    prev_i = jnp.maximum(lanes - _I(1), _I(0))
    next_i = jnp.minimum(lanes + _I(1), _I(15))

    def block(k, carry):
        off = i0 + jnp.minimum(k * _I(QB), cnt - _I(QB))

        pltpu.sync_copy(xyz_hbm.at[pl.ds(off, QB)], xv)
        pltpu.sync_copy(xyz_hbm.at[pl.ds(off + _I(N), QB)], yv)
        pltpu.sync_copy(xyz_hbm.at[pl.ds(off + _I(2 * N), QB)], zv)

        def vec(v, c2):
            s = v * _I(16)
            h = (
                xv[pl.ds(s, 16)] * p0
                + yv[pl.ds(s, 16)] * p1
                + zv[pl.ds(s, 16)] * p2
            ) & hm
            hv[pl.ds(s, 16)] = h
            sk = jnp.sort((h << _I(4)) | lanes)
            ow = sk >> _I(19)
            prev = sk.at[prev_i].get(mode="promise_in_bounds") >> _I(19)
            nxt = sk.at[next_i].get(mode="promise_in_bounds") >> _I(19)
            chg = (ow != prev) | (lanes == _I(0))
            re = (ow != nxt) | (lanes == _I(15))
            runstart = plsc.cummax(jnp.where(chg, lanes, _I(0)))
            rank = lanes - runstart
            cntow = plsc.load_gather(cnt_v, [ow])
            addr = ow * _I(SEG_CAP) + cntow + rank
            plsc.store_scatter(segh_v, [addr], sk >> _I(4))
            plsc.store_scatter(segj_v, [addr], off + s + (sk & _I(15)))
            plsc.addupdate_scatter(cnt_v, [ow], rank + _I(1), mask=re)
            return c2

        lax.fori_loop(_I(0), _I(VPB), vec, _I(0))
        pltpu.sync_copy(hv, h_hbm.at[pl.ds(off, QB)])
        return carry

    lax.fori_loop(_I(0), trips, block, _I(0))

    cps = []
    for o in range(NW):
        dst = _I(o * SEGW) + wid * _I(SEG_CAP)
        cps.append(pltpu.async_copy(
            segh_v.at[pl.ds(_I(o * SEG_CAP), SEG_CAP)],
            segh_hbm.at[pl.ds(dst, SEG_CAP)], sem))
        cps.append(pltpu.async_copy(
            segj_v.at[pl.ds(_I(o * SEG_CAP), SEG_CAP)],
            segj_hbm.at[pl.ds(dst, SEG_CAP)], sem))
    for c in cps:
        c.wait()


UNROLL = 8


@functools.partial(
    pl.kernel,
    out_type=jax.ShapeDtypeStruct((TABLE,), jnp.int32),
    mesh=_mesh,
    compiler_params=_params,
    scratch_types=[
        pltpu.VMEM((SEGW,), jnp.int32),
        pltpu.VMEM((SEGW,), jnp.int32),
        pltpu.VMEM((BPW,), jnp.int32),
        pltpu.SemaphoreType.DMA,
    ],
)
def _build_k(segh_hbm, segj_hbm, win_hbm, sgh_v, sgj_v, win_v, sem):
    wid = _wid()
    base = wid * _I(BPW)
    zeros = jnp.zeros((16,), jnp.int32)

    def zinit(k, carry):
        win_v[pl.ds(k * _I(16), 16)] = zeros
        return carry

    lax.fori_loop(_I(0), _I(BPW // 16), zinit, _I(0))

    slab = wid * _I(SEGW)
    c0 = pltpu.async_copy(segh_hbm.at[pl.ds(slab, SEGW)], sgh_v, sem)
    c1 = pltpu.async_copy(segj_hbm.at[pl.ds(slab, SEGW)], sgj_v, sem)
    c0.wait()
    c1.wait()

    ubpw = jnp.full((16,), BPW, jnp.uint32)

    def vec(v, carry):
        s = v * _I(16 * UNROLL)
        for u in range(UNROLL):
            sl = pl.ds(s + u * 16, 16)
            idx = sgh_v[sl] - base
            m = plsc.bitcast(idx, jnp.uint32) < ubpw
            idxc = jnp.where(m, idx, _I(0))
            plsc.store_scatter(win_v, [idxc], sgj_v[sl], mask=m)
        return carry

    lax.fori_loop(_I(0), _I(SEGW // (16 * UNROLL)), vec, _I(0))
    pltpu.sync_copy(win_v, win_hbm.at[pl.ds(base, BPW)])


@functools.partial(
    pl.kernel,
    out_type=jax.ShapeDtypeStruct((N, D), jnp.float32),
    mesh=_mesh,
    compiler_params=_params,
    scratch_types=[
        pltpu.VMEM((QB,), jnp.int32),
        pltpu.VMEM((QB,), jnp.int32),
        pltpu.VMEM((QB, D), jnp.float32),
        pltpu.SemaphoreType.DMA,
    ],
)
def _query_k(h_hbm, win_hbm, feat_hbm, out_hbm, hv, gv, rows_v, sem):
    wid = _wid()
    i0, cnt = _tile_range(wid)
    trips = (cnt + _I(QB - 1)) // _I(QB)

    def block(k, carry):
        off = i0 + jnp.minimum(k * _I(QB), cnt - _I(QB))
        pltpu.sync_copy(h_hbm.at[pl.ds(off, QB)], hv)
        pltpu.async_copy(win_hbm.at[hv], gv, sem).wait()
        pltpu.async_copy(feat_hbm.at[gv], rows_v, sem).wait()
        pltpu.sync_copy(rows_v, out_hbm.at[pl.ds(off, QB)])
        return carry

    lax.fori_loop(_I(0), trips, block, _I(0))


_params_tc = pltpu.CompilerParams(
    needs_layout_passes=False, use_tc_tiling_on_sc=True
)
RB = 4
NCOL = N // 128
TAILR = N - NCOL * 128


@functools.partial(
    pl.kernel,
    out_type=jax.ShapeDtypeStruct((D, N), jnp.float32),
    mesh=_mesh,
    compiler_params=_params_tc,
    scratch_types=(
        [pltpu.VMEM((RB * 128, D), jnp.float32)]
        + [pltpu.VMEM((8, 128), jnp.float32) for _ in range(2 * RB)]
        + [pltpu.VMEM((8, 32), jnp.float32) for _ in range(2)]
        + [pltpu.SemaphoreType.DMA, pltpu.SemaphoreType.DMA]
    ),
)
def _fmt_k(lin_hbm, out_hbm, rv, *rest):
    pv = rest[: 2 * RB]
    pt0, pt1, rsem, wsem = rest[2 * RB:]
    wid = _wid()
    cstart = wid * _I(122) + jnp.minimum(wid, _I(2))
    ccnt = jnp.where(wid < _I(2), _I(123), _I(122))
    nb = (ccnt + _I(RB - 1)) // _I(RB)
    lanes = lax.iota(jnp.int32, 16)
    cf = [jnp.full((16,), f, jnp.int32) for f in range(D)]

    def batch(b, carry):
        cb = cstart + jnp.minimum(b * _I(RB), ccnt - _I(RB))
        for u in range(RB):
            pltpu.async_copy(
                lin_hbm.at[pl.ds((cb + _I(u)) * _I(128), 128), :],
                rv.at[pl.ds(_I(u * 128), 128), :], rsem)
        for u in range(RB):
            pltpu.make_async_copy(
                lin_hbm.at[pl.ds(_I(0), 128), :],
                rv.at[pl.ds(_I(0), 128), :], rsem).wait()
        for u in range(RB):
            for g in range(8):
                ic = lanes + _I(u * 128 + g * 16)
                for t in range(D):
                    val = plsc.load_gather(rv, [ic, cf[t]])
                    pv[2 * u + t // 8][_I(t % 8), pl.ds(g * 16, 16)] = val
        for u in range(RB):
            for pp in range(2):
                pltpu.async_copy(
                    pv[2 * u + pp],
                    out_hbm.at[pl.ds(_I(pp * 8), 8),
                               pl.ds((cb + _I(u)) * _I(128), 128)], wsem)
        for u in range(2 * RB):
            pltpu.make_async_copy(
                pv[0], out_hbm.at[pl.ds(_I(0), 8), pl.ds(_I(0), 128)],
                wsem).wait()
        return carry

    lax.fori_loop(_I(0), nb, batch, _I(0))

    @pl.when(wid == _I(31))
    def _tail():
        pltpu.async_copy(
            lin_hbm.at[pl.ds(_I(NCOL * 128), TAILR), :],
            rv.at[pl.ds(_I(0), TAILR), :], rsem)
        pltpu.make_async_copy(
            lin_hbm.at[pl.ds(_I(0), TAILR), :],
            rv.at[pl.ds(_I(0), TAILR), :], rsem).wait()
        pt = [pt0, pt1]
        for g in range(TAILR // 16):
            ic = lanes + _I(g * 16)
            for t in range(D):
                val = plsc.load_gather(rv, [ic, cf[t]])
                pt[t // 8][_I(t % 8), pl.ds(g * 16, 16)] = val
        for pp in range(2):
            pltpu.async_copy(
                pt[pp],
                out_hbm.at[pl.ds(_I(pp * 8), 8),
                           pl.ds(_I(NCOL * 128), TAILR)], wsem)
        for pp in range(2):
            pltpu.make_async_copy(
                pt0, out_hbm.at[pl.ds(_I(0), 8),
                                pl.ds(_I(NCOL * 128), TAILR)], wsem).wait()


def kernel(coords, features):
    xyz = coords.T.astype(jnp.int32).reshape(3 * N)
    h, segh, segj = _route_k(xyz)
    win = _build_k(segh, segj)
    lin = _query_k(h, win, features)
    return _fmt_k(lin).T


# speedup vs baseline: 1.3533x; 1.3533x over previous
import functools

import jax
import jax.numpy as jnp
from jax import lax
from jax.experimental import pallas as pl
from jax.experimental.pallas import tpu as pltpu
from jax.experimental.pallas import tpu_sc as plsc

P0, P1, P2 = 73856093, 19349663, 83492791
TABLE = 1 << 20
N = 500000
D = 16

NW = 32
BPW = TABLE // NW
QB = 2048
VPB = QB // 16
CPT = 15632
HMASK = TABLE - 1

_mesh = plsc.VectorSubcoreMesh(core_axis_name="c", subcore_axis_name="s")
_params = pltpu.CompilerParams(
    needs_layout_passes=False, use_tc_tiling_on_sc=False
)


_I = jnp.int32


def _wid():
    return lax.axis_index("s") * _I(2) + lax.axis_index("c")


def _tile_range(wid):
    i0 = wid * _I(CPT)
    cnt = jnp.minimum(_I(CPT), _I(N) - i0)
    return i0, cnt


SEG_CAP = 1024
SEGW = NW * SEG_CAP
SLACK = 2048
SENT = 0x7FFFFFFF


@functools.partial(
    pl.kernel,
    out_type=[
        jax.ShapeDtypeStruct((N,), jnp.int32),
        jax.ShapeDtypeStruct((NW * SEGW,), jnp.int32),
        jax.ShapeDtypeStruct((NW * SEGW,), jnp.int32),
    ],
    mesh=_mesh,
    compiler_params=_params,
    scratch_types=[
        pltpu.VMEM((QB,), jnp.int32),
        pltpu.VMEM((QB,), jnp.int32),
        pltpu.VMEM((QB,), jnp.int32),
        pltpu.VMEM((QB,), jnp.int32),
        pltpu.VMEM((SEGW + SLACK,), jnp.int32),
        pltpu.VMEM((SEGW + SLACK,), jnp.int32),
        pltpu.VMEM((32,), jnp.int32),
        pltpu.SemaphoreType.DMA,
    ],
)
def _route_k(xyz_hbm, h_hbm, segh_hbm, segj_hbm,
             xv, yv, zv, hv, segh_v, segj_v, cnt_v, sem):
    wid = _wid()
    i0, cnt = _tile_range(wid)
    trips = (cnt + _I(QB - 1)) // _I(QB)
    lanes = lax.iota(jnp.int32, 16)

    sent = jnp.full((16,), SENT, jnp.int32)

    def sinit(k, carry):
        segh_v[pl.ds(k * _I(16), 16)] = sent
        return carry

    lax.fori_loop(_I(0), _I((SEGW + SLACK) // 16), sinit, _I(0))
    cnt_v[pl.ds(_I(0), 16)] = jnp.zeros((16,), jnp.int32)
    cnt_v[pl.ds(_I(16), 16)] = jnp.zeros((16,), jnp.int32)

    p0 = jnp.full((16,), P0, jnp.int32)
    p1 = jnp.full((16,), P1, jnp.int32)
    p2 = jnp.full((16,), P2, jnp.int32)
    hm = jnp.full((16,), HMASK, jnp.int32)
    prev_i = jnp.maximum(lanes - _I(1), _I(0))
    next_i = jnp.minimum(lanes + _I(1), _I(15))

    def block(k, carry):
        off = i0 + jnp.minimum(k * _I(QB), cnt - _I(QB))

        pltpu.sync_copy(xyz_hbm.at[pl.ds(off, QB)], xv)
        pltpu.sync_copy(xyz_hbm.at[pl.ds(off + _I(N), QB)], yv)
        pltpu.sync_copy(xyz_hbm.at[pl.ds(off + _I(2 * N), QB)], zv)

        def vec(v, c2):
            s = v * _I(16)
            h = (
                xv[pl.ds(s, 16)] * p0
                + yv[pl.ds(s, 16)] * p1
                + zv[pl.ds(s, 16)] * p2
            ) & hm
            hv[pl.ds(s, 16)] = h
            sk = jnp.sort((h << _I(4)) | lanes)
            ow = sk >> _I(19)
            prev = sk.at[prev_i].get(mode="promise_in_bounds") >> _I(19)
            nxt = sk.at[next_i].get(mode="promise_in_bounds") >> _I(19)
            chg = (ow != prev) | (lanes == _I(0))
            re = (ow != nxt) | (lanes == _I(15))
            runstart = plsc.cummax(jnp.where(chg, lanes, _I(0)))
            rank = lanes - runstart
            cntow = plsc.load_gather(cnt_v, [ow])
            addr = ow * _I(SEG_CAP) + cntow + rank
            plsc.store_scatter(segh_v, [addr], sk >> _I(4))
            plsc.store_scatter(segj_v, [addr], off + s + (sk & _I(15)))
            plsc.addupdate_scatter(cnt_v, [ow], rank + _I(1), mask=re)
            return c2

        lax.fori_loop(_I(0), _I(VPB), vec, _I(0))
        pltpu.sync_copy(hv, h_hbm.at[pl.ds(off, QB)])
        return carry

    lax.fori_loop(_I(0), trips, block, _I(0))

    cps = []
    for o in range(NW):
        dst = _I(o * SEGW) + wid * _I(SEG_CAP)
        cps.append(pltpu.async_copy(
            segh_v.at[pl.ds(_I(o * SEG_CAP), SEG_CAP)],
            segh_hbm.at[pl.ds(dst, SEG_CAP)], sem))
        cps.append(pltpu.async_copy(
            segj_v.at[pl.ds(_I(o * SEG_CAP), SEG_CAP)],
            segj_hbm.at[pl.ds(dst, SEG_CAP)], sem))
    for c in cps:
        c.wait()


UNROLL = 8


@functools.partial(
    pl.kernel,
    out_type=jax.ShapeDtypeStruct((TABLE,), jnp.int32),
    mesh=_mesh,
    compiler_params=_params,
    scratch_types=[
        pltpu.VMEM((SEGW,), jnp.int32),
        pltpu.VMEM((SEGW,), jnp.int32),
        pltpu.VMEM((BPW,), jnp.int32),
        pltpu.SemaphoreType.DMA,
    ],
)
def _build_k(segh_hbm, segj_hbm, win_hbm, sgh_v, sgj_v, win_v, sem):
    wid = _wid()
    base = wid * _I(BPW)
    zeros = jnp.zeros((16,), jnp.int32)

    def zinit(k, carry):
        win_v[pl.ds(k * _I(16), 16)] = zeros
        return carry

    lax.fori_loop(_I(0), _I(BPW // 16), zinit, _I(0))

    slab = wid * _I(SEGW)
    c0 = pltpu.async_copy(segh_hbm.at[pl.ds(slab, SEGW)], sgh_v, sem)
    c1 = pltpu.async_copy(segj_hbm.at[pl.ds(slab, SEGW)], sgj_v, sem)
    c0.wait()
    c1.wait()

    ubpw = jnp.full((16,), BPW, jnp.uint32)

    def vec(v, carry):
        s = v * _I(16 * UNROLL)
        for u in range(UNROLL):
            sl = pl.ds(s + u * 16, 16)
            idx = sgh_v[sl] - base
            m = plsc.bitcast(idx, jnp.uint32) < ubpw
            idxc = jnp.where(m, idx, _I(0))
            plsc.store_scatter(win_v, [idxc], sgj_v[sl], mask=m)
        return carry

    lax.fori_loop(_I(0), _I(SEGW // (16 * UNROLL)), vec, _I(0))
    pltpu.sync_copy(win_v, win_hbm.at[pl.ds(base, BPW)])


@functools.partial(
    pl.kernel,
    out_type=jax.ShapeDtypeStruct((N, D), jnp.float32),
    mesh=_mesh,
    compiler_params=_params,
    scratch_types=[
        pltpu.VMEM((QB,), jnp.int32),
        pltpu.VMEM((QB,), jnp.int32),
        pltpu.VMEM((QB, D), jnp.float32),
        pltpu.SemaphoreType.DMA,
    ],
)
def _query_k(h_hbm, win_hbm, feat_hbm, out_hbm, hv, gv, rows_v, sem):
    wid = _wid()
    i0, cnt = _tile_range(wid)
    trips = (cnt + _I(QB - 1)) // _I(QB)

    def block(k, carry):
        off = i0 + jnp.minimum(k * _I(QB), cnt - _I(QB))
        pltpu.sync_copy(h_hbm.at[pl.ds(off, QB)], hv)
        pltpu.async_copy(win_hbm.at[hv], gv, sem).wait()
        pltpu.async_copy(feat_hbm.at[gv], rows_v, sem).wait()
        pltpu.sync_copy(rows_v, out_hbm.at[pl.ds(off, QB)])
        return carry

    lax.fori_loop(_I(0), trips, block, _I(0))


def kernel(coords, features):
    xyz = coords.T.astype(jnp.int32).reshape(3 * N)
    h, segh, segj = _route_k(xyz)
    win = _build_k(segh, segj)
    return _query_k(h, win, features)
